# baseline (speedup 1.0000x reference)
import functools

import jax
import jax.numpy as jnp
from jax import lax
from jax.experimental import pallas as pl
from jax.experimental.pallas import tpu as pltpu
from jax.experimental.pallas import tpu_sc as plsc

H, S, D = 12, 2048, 64
NUM_LAYERS = 24
ROWS = H * D
ROW_CHUNK = 32

VOCAB = 100000
NWORK = 32
LANES = 16
VPAD = 100352
CHUNK = VPAD // NWORK
NVEC = CHUNK // LANES
NEG = -3.4e38


def _copy_merge_body(*refs):
    ins = refs[:NUM_LAYERS]
    vals_ref, idxs_ref, mxse_ref, save_id_ref, bz_ref = (
        refs[NUM_LAYERS:NUM_LAYERS + 5])
    outs = refs[NUM_LAYERS + 5:2 * NUM_LAYERS + 5]
    save_out_ref, prob_ref, ids_ref, max_ref = refs[2 * NUM_LAYERS + 5:]
    for k in range(NUM_LAYERS):
        outs[k][...] = jnp.broadcast_to(ins[k][...][None], outs[k].shape)

    @pl.when(pl.program_id(0) == 0)
    def _():
        _merge_body(vals_ref, idxs_ref, mxse_ref, save_id_ref, bz_ref,
                    save_out_ref, prob_ref, ids_ref, max_ref)


def _sc_topk_body(logits_ref, vals_ref, idxs_ref, mxse_ref,
                  buf, vbuf, ibuf, mbuf):
    c = lax.axis_index("c")
    s = lax.axis_index("s")
    wid = s * 2 + c
    base = wid * CHUNK
    pltpu.sync_copy(logits_ref.at[pl.ds(base, CHUNK)], buf)
    lanes = lax.iota(jnp.int32, LANES)

    def step(j, carry):
        m1, i1, m2, i2, m3, i3, mx, se = carry
        x = buf[pl.ds(j * LANES, LANES)]
        idx = base + j * LANES + lanes
        nm = jnp.maximum(mx, x)
        se = se * jnp.exp(mx - nm) + jnp.exp(x - nm)
        mx = nm
        b1 = x > m1
        t1v = jnp.where(b1, m1, x)
        t1i = jnp.where(b1, i1, idx)
        m1 = jnp.where(b1, x, m1)
        i1 = jnp.where(b1, idx, i1)
        b2 = t1v > m2
        t2v = jnp.where(b2, m2, t1v)
        t2i = jnp.where(b2, i2, t1i)
        m2 = jnp.where(b2, t1v, m2)
        i2 = jnp.where(b2, t1i, i2)
        b3 = t2v > m3
        m3 = jnp.where(b3, t2v, m3)
        i3 = jnp.where(b3, t2i, i3)
        return m1, i1, m2, i2, m3, i3, mx, se

    neg = jnp.full((LANES,), NEG, jnp.float32)
    zero_i = jnp.zeros((LANES,), jnp.int32)
    zero_f = jnp.zeros((LANES,), jnp.float32)
    m1, i1, m2, i2, m3, i3, mx, se = lax.fori_loop(
        0, NVEC, step, (neg, zero_i, neg, zero_i, neg, zero_i, neg, zero_f))

    vbuf[pl.ds(0, LANES)] = m1
    vbuf[pl.ds(LANES, LANES)] = m2
    vbuf[pl.ds(2 * LANES, LANES)] = m3
    ibuf[pl.ds(0, LANES)] = i1
    ibuf[pl.ds(LANES, LANES)] = i2
    ibuf[pl.ds(2 * LANES, LANES)] = i3
    mbuf[pl.ds(0, LANES)] = mx
    mbuf[pl.ds(LANES, LANES)] = se
    pltpu.sync_copy(vbuf, vals_ref.at[wid])
    pltpu.sync_copy(ibuf, idxs_ref.at[wid])
    pltpu.sync_copy(mbuf, mxse_ref.at[wid])


@functools.lru_cache(maxsize=1)
def _sc_topk():
    return pl.kernel(
        _sc_topk_body,
        mesh=plsc.VectorSubcoreMesh(core_axis_name="c",
                                    subcore_axis_name="s"),
        out_type=[
            jax.ShapeDtypeStruct((NWORK, 3 * LANES), jnp.float32),
            jax.ShapeDtypeStruct((NWORK, 3 * LANES), jnp.int32),
            jax.ShapeDtypeStruct((NWORK, 2 * LANES), jnp.float32),
        ],
        scratch_types=[
            pltpu.VMEM((CHUNK,), jnp.float32),
            pltpu.VMEM((3 * LANES,), jnp.float32),
            pltpu.VMEM((3 * LANES,), jnp.int32),
            pltpu.VMEM((2 * LANES,), jnp.float32),
        ],
    )


def _merge_body(vals_ref, idxs_ref, mxse_ref, save_id_ref, bz_ref,
                save_out_ref, prob_ref, ids_ref, max_ref):
    v = vals_ref[...]
    ia = idxs_ref[...]
    mx = mxse_ref[:, 0:LANES]
    se = mxse_ref[:, LANES:2 * LANES]
    m = jnp.max(mx)
    ssum = jnp.sum(se * jnp.exp(mx - m))
    lse = jnp.log(ssum) + m
    neg_inf = jnp.float32(-jnp.inf)
    cur = v
    vals, idxs = [], []
    for _ in range(save_id_ref.shape[0]):
        vj = jnp.max(cur)
        hit = cur >= vj
        ij = jnp.min(jnp.where(hit, ia, VOCAB))
        vals.append(vj.reshape(1, 1))
        idxs.append(ij.reshape(1, 1))
        cur = jnp.where(hit & (ia == ij), neg_inf, cur)
    ids_col = jnp.concatenate(idxs, axis=0) + bz_ref[0, 0]
    prob_col = jnp.concatenate(vals, axis=0) - lse
    ids_ref[...] = ids_col
    prob_ref[...] = prob_col
    save_out_ref[:, 0:1] = save_id_ref[...]
    save_out_ref[:, 1:2] = ids_col
    max_ref[...] = ids_col[0:1, :]


def kernel(kv0, kv1, kv2, kv3, kv4, kv5, kv6, kv7, kv8, kv9, kv10, kv11,
           kv12, kv13, kv14, kv15, kv16, kv17, kv18, kv19, kv20, kv21,
           kv22, kv23, logits, save_id, beam_size):
    kvs = [kv0, kv1, kv2, kv3, kv4, kv5, kv6, kv7, kv8, kv9, kv10, kv11,
           kv12, kv13, kv14, kv15, kv16, kv17, kv18, kv19, kv20, kv21,
           kv22, kv23]
    beam = save_id.shape[0]

    logits_padded = jnp.concatenate(
        [logits.reshape(-1),
         jnp.full((VPAD - VOCAB,), NEG, jnp.float32)])
    sc_vals, sc_idxs, sc_mxse = _sc_topk()(logits_padded)

    flat_kvs = [kv.reshape(H, S, D).swapaxes(1, 2).reshape(ROWS, S)
                for kv in kvs]
    grid = (ROWS // ROW_CHUNK,)
    in_spec = pl.BlockSpec((ROW_CHUNK, S), lambda i: (i, 0))
    out_spec = pl.BlockSpec((beam, ROW_CHUNK, S), lambda i: (0, i, 0))
    bz = (jnp.asarray(beam_size, jnp.int32) - jnp.int32(beam)).reshape(1, 1)
    outputs = pl.pallas_call(
        _copy_merge_body,
        grid=grid,
        in_specs=[in_spec] * NUM_LAYERS + [
            pl.BlockSpec((NWORK, 3 * LANES), lambda i: (0, 0)),
            pl.BlockSpec((NWORK, 3 * LANES), lambda i: (0, 0)),
            pl.BlockSpec((NWORK, 2 * LANES), lambda i: (0, 0)),
            pl.BlockSpec((beam, 1), lambda i: (0, 0)),
            pl.BlockSpec((1, 1), lambda i: (0, 0)),
        ],
        out_specs=[out_spec] * NUM_LAYERS + [
            pl.BlockSpec((beam, 2), lambda i: (0, 0)),
            pl.BlockSpec((beam, 1), lambda i: (0, 0)),
            pl.BlockSpec((beam, 1), lambda i: (0, 0)),
            pl.BlockSpec((1, 1), lambda i: (0, 0)),
        ],
        out_shape=[jax.ShapeDtypeStruct((beam, ROWS, S), kv.dtype)
                   for kv in kvs] + [
            jax.ShapeDtypeStruct((beam, 2), jnp.int32),
            jax.ShapeDtypeStruct((beam, 1), jnp.float32),
            jax.ShapeDtypeStruct((beam, 1), jnp.int32),
            jax.ShapeDtypeStruct((1, 1), jnp.int32),
        ],
    )(*flat_kvs, sc_vals, sc_idxs, sc_mxse, save_id, bz)
    tiled = [t.reshape(beam, H, D, S).swapaxes(2, 3)
             for t in outputs[:NUM_LAYERS]]
    save_out, prob, ids, max_idx = outputs[NUM_LAYERS:]

    return (*tiled, save_out, prob, ids, max_idx)


# speedup vs baseline: 1.0182x; 1.0182x over previous
import functools

import jax
import jax.numpy as jnp
from jax import lax
from jax.experimental import pallas as pl
from jax.experimental.pallas import tpu as pltpu
from jax.experimental.pallas import tpu_sc as plsc

H, S, D = 12, 2048, 64
NUM_LAYERS = 24
ROWS = H * D
ROW_CHUNK = 32

VOCAB = 100000
NWORK = 32
LANES = 16
VPAD = 100352
CHUNK = VPAD // NWORK
NVEC = CHUNK // LANES
NEG = -3.4e38


def _copy_merge_body(*refs):
    ins = refs[:NUM_LAYERS]
    vals_ref, idxs_ref, mxse_ref, save_id_ref, bz_ref = (
        refs[NUM_LAYERS:NUM_LAYERS + 5])
    outs = refs[NUM_LAYERS + 5:2 * NUM_LAYERS + 5]
    save_out_ref, prob_ref, ids_ref, max_ref = refs[2 * NUM_LAYERS + 5:]
    for k in range(NUM_LAYERS):
        outs[k][...] = jnp.broadcast_to(ins[k][...][None], outs[k].shape)

    @pl.when(pl.program_id(0) == 0)
    def _():
        _merge_body(vals_ref, idxs_ref, mxse_ref, save_id_ref, bz_ref,
                    save_out_ref, prob_ref, ids_ref, max_ref)


def _sc_topk_body(logits_ref, vals_ref, idxs_ref, mxse_ref,
                  buf, vbuf, ibuf, mbuf):
    c = lax.axis_index("c")
    s = lax.axis_index("s")
    wid = s * 2 + c
    base = wid * CHUNK
    pltpu.sync_copy(logits_ref.at[pl.ds(base, CHUNK)], buf)
    lanes = lax.iota(jnp.int32, LANES)

    def step(j, carry):
        m1, i1, m2, i2, m3, i3, mx, se = carry
        x = buf[pl.ds(j * LANES, LANES)]
        idx = base + j * LANES + lanes
        nm = jnp.maximum(mx, x)
        se = se * jnp.exp(mx - nm) + jnp.exp(x - nm)
        mx = nm
        b1 = x > m1
        t1v = jnp.where(b1, m1, x)
        t1i = jnp.where(b1, i1, idx)
        m1 = jnp.where(b1, x, m1)
        i1 = jnp.where(b1, idx, i1)
        b2 = t1v > m2
        t2v = jnp.where(b2, m2, t1v)
        t2i = jnp.where(b2, i2, t1i)
        m2 = jnp.where(b2, t1v, m2)
        i2 = jnp.where(b2, t1i, i2)
        b3 = t2v > m3
        m3 = jnp.where(b3, t2v, m3)
        i3 = jnp.where(b3, t2i, i3)
        return m1, i1, m2, i2, m3, i3, mx, se

    neg = jnp.full((LANES,), NEG, jnp.float32)
    zero_i = jnp.zeros((LANES,), jnp.int32)
    zero_f = jnp.zeros((LANES,), jnp.float32)
    m1, i1, m2, i2, m3, i3, mx, se = lax.fori_loop(
        0, NVEC, step, (neg, zero_i, neg, zero_i, neg, zero_i, neg, zero_f))

    vbuf[pl.ds(0, LANES)] = m1
    vbuf[pl.ds(LANES, LANES)] = m2
    vbuf[pl.ds(2 * LANES, LANES)] = m3
    ibuf[pl.ds(0, LANES)] = i1
    ibuf[pl.ds(LANES, LANES)] = i2
    ibuf[pl.ds(2 * LANES, LANES)] = i3
    mbuf[pl.ds(0, LANES)] = mx
    mbuf[pl.ds(LANES, LANES)] = se
    pltpu.sync_copy(vbuf, vals_ref.at[wid])
    pltpu.sync_copy(ibuf, idxs_ref.at[wid])
    pltpu.sync_copy(mbuf, mxse_ref.at[wid])


@functools.lru_cache(maxsize=1)
def _sc_topk():
    return pl.kernel(
        _sc_topk_body,
        mesh=plsc.VectorSubcoreMesh(core_axis_name="c",
                                    subcore_axis_name="s"),
        out_type=[
            jax.ShapeDtypeStruct((NWORK, 3 * LANES), jnp.float32),
            jax.ShapeDtypeStruct((NWORK, 3 * LANES), jnp.int32),
            jax.ShapeDtypeStruct((NWORK, 2 * LANES), jnp.float32),
        ],
        scratch_types=[
            pltpu.VMEM((CHUNK,), jnp.float32),
            pltpu.VMEM((3 * LANES,), jnp.float32),
            pltpu.VMEM((3 * LANES,), jnp.int32),
            pltpu.VMEM((2 * LANES,), jnp.float32),
        ],
    )


def _merge_body(vals_ref, idxs_ref, mxse_ref, save_id_ref, bz_ref,
                save_out_ref, prob_ref, ids_ref, max_ref):
    v = vals_ref[...]
    ia = idxs_ref[...]
    mx = mxse_ref[:, 0:LANES]
    se = mxse_ref[:, LANES:2 * LANES]
    m = jnp.max(mx)
    ssum = jnp.sum(se * jnp.exp(mx - m))
    lse = jnp.log(ssum) + m
    neg_inf = jnp.float32(-jnp.inf)
    cur = v
    vals, idxs = [], []
    for _ in range(save_id_ref.shape[1]):
        vj = jnp.max(cur)
        hit = cur >= vj
        ij = jnp.min(jnp.where(hit, ia, VOCAB))
        vals.append(vj.reshape(1, 1))
        idxs.append(ij.reshape(1, 1))
        cur = jnp.where(hit & (ia == ij), neg_inf, cur)
    ids_row = jnp.concatenate(idxs, axis=1) + bz_ref[0, 0]
    prob_row = jnp.concatenate(vals, axis=1) - lse
    ids_ref[...] = ids_row
    prob_ref[...] = prob_row
    save_out_ref[0:1, :] = save_id_ref[...]
    save_out_ref[1:2, :] = ids_row
    max_ref[...] = ids_row[:, 0:1]


def kernel(kv0, kv1, kv2, kv3, kv4, kv5, kv6, kv7, kv8, kv9, kv10, kv11,
           kv12, kv13, kv14, kv15, kv16, kv17, kv18, kv19, kv20, kv21,
           kv22, kv23, logits, save_id, beam_size):
    kvs = [kv0, kv1, kv2, kv3, kv4, kv5, kv6, kv7, kv8, kv9, kv10, kv11,
           kv12, kv13, kv14, kv15, kv16, kv17, kv18, kv19, kv20, kv21,
           kv22, kv23]
    beam = save_id.shape[0]

    logits_padded = jnp.concatenate(
        [logits.reshape(-1),
         jnp.full((VPAD - VOCAB,), NEG, jnp.float32)])
    sc_vals, sc_idxs, sc_mxse = _sc_topk()(logits_padded)

    flat_kvs = [kv.reshape(H, S, D).swapaxes(1, 2).reshape(ROWS, S)
                for kv in kvs]
    grid = (ROWS // ROW_CHUNK,)
    in_spec = pl.BlockSpec((ROW_CHUNK, S), lambda i: (i, 0))
    out_spec = pl.BlockSpec((beam, ROW_CHUNK, S), lambda i: (0, i, 0))
    bz = (jnp.asarray(beam_size, jnp.int32) - jnp.int32(beam)).reshape(1, 1)
    outputs = pl.pallas_call(
        _copy_merge_body,
        grid=grid,
        in_specs=[in_spec] * NUM_LAYERS + [
            pl.BlockSpec((NWORK, 3 * LANES), lambda i: (0, 0)),
            pl.BlockSpec((NWORK, 3 * LANES), lambda i: (0, 0)),
            pl.BlockSpec((NWORK, 2 * LANES), lambda i: (0, 0)),
            pl.BlockSpec((1, beam), lambda i: (0, 0)),
            pl.BlockSpec((1, 1), lambda i: (0, 0)),
        ],
        out_specs=[out_spec] * NUM_LAYERS + [
            pl.BlockSpec((2, beam), lambda i: (0, 0)),
            pl.BlockSpec((1, beam), lambda i: (0, 0)),
            pl.BlockSpec((1, beam), lambda i: (0, 0)),
            pl.BlockSpec((1, 1), lambda i: (0, 0)),
        ],
        out_shape=[jax.ShapeDtypeStruct((beam, ROWS, S), kv.dtype)
                   for kv in kvs] + [
            jax.ShapeDtypeStruct((2, beam), jnp.int32),
            jax.ShapeDtypeStruct((1, beam), jnp.float32),
            jax.ShapeDtypeStruct((1, beam), jnp.int32),
            jax.ShapeDtypeStruct((1, 1), jnp.int32),
        ],
    )(*flat_kvs, sc_vals, sc_idxs, sc_mxse, save_id.T, bz)
    tiled = [t.reshape(beam, H, D, S).swapaxes(2, 3)
             for t in outputs[:NUM_LAYERS]]
    save_out_t, prob_t, ids_t, max_idx = outputs[NUM_LAYERS:]

    return (*tiled, save_out_t.T, prob_t.T, ids_t.T, max_idx)


# speedup vs baseline: 1.0287x; 1.0103x over previous
import functools

import jax
import jax.numpy as jnp
from jax import lax
from jax.experimental import pallas as pl
from jax.experimental.pallas import tpu as pltpu
from jax.experimental.pallas import tpu_sc as plsc

H, S, D = 12, 2048, 64
NUM_LAYERS = 24
ROWS = H * D
ROW_CHUNK = 32

VOCAB = 100000
NWORK = 32
LANES = 16
VPAD = 100352
CHUNK = VPAD // NWORK
NVEC = CHUNK // LANES
NEG = -3.4e38


def _copy_body(*refs):
    ins = refs[:NUM_LAYERS]
    outs = refs[NUM_LAYERS:]
    for k in range(NUM_LAYERS):
        outs[k][...] = jnp.broadcast_to(ins[k][...][None], outs[k].shape)


def _sc_topk_body(logits_ref, vals_ref, idxs_ref, mxse_ref,
                  buf, vbuf, ibuf, mbuf):
    c = lax.axis_index("c")
    s = lax.axis_index("s")
    wid = s * 2 + c
    base = wid * CHUNK
    pltpu.sync_copy(logits_ref.at[pl.ds(base, CHUNK)], buf)
    lanes = lax.iota(jnp.int32, LANES)

    def step(j, carry):
        m1, i1, m2, i2, m3, i3, mx, se = carry
        x = buf[pl.ds(j * LANES, LANES)]
        idx = base + j * LANES + lanes
        nm = jnp.maximum(mx, x)
        se = se * jnp.exp(mx - nm) + jnp.exp(x - nm)
        mx = nm
        b1 = x > m1
        t1v = jnp.where(b1, m1, x)
        t1i = jnp.where(b1, i1, idx)
        m1 = jnp.where(b1, x, m1)
        i1 = jnp.where(b1, idx, i1)
        b2 = t1v > m2
        t2v = jnp.where(b2, m2, t1v)
        t2i = jnp.where(b2, i2, t1i)
        m2 = jnp.where(b2, t1v, m2)
        i2 = jnp.where(b2, t1i, i2)
        b3 = t2v > m3
        m3 = jnp.where(b3, t2v, m3)
        i3 = jnp.where(b3, t2i, i3)
        return m1, i1, m2, i2, m3, i3, mx, se

    neg = jnp.full((LANES,), NEG, jnp.float32)
    zero_i = jnp.zeros((LANES,), jnp.int32)
    zero_f = jnp.zeros((LANES,), jnp.float32)
    m1, i1, m2, i2, m3, i3, mx, se = lax.fori_loop(
        0, NVEC, step, (neg, zero_i, neg, zero_i, neg, zero_i, neg, zero_f))

    vbuf[pl.ds(0, LANES)] = m1
    vbuf[pl.ds(LANES, LANES)] = m2
    vbuf[pl.ds(2 * LANES, LANES)] = m3
    ibuf[pl.ds(0, LANES)] = i1
    ibuf[pl.ds(LANES, LANES)] = i2
    ibuf[pl.ds(2 * LANES, LANES)] = i3
    mbuf[pl.ds(0, LANES)] = mx
    mbuf[pl.ds(LANES, LANES)] = se
    pltpu.sync_copy(vbuf, vals_ref.at[wid])
    pltpu.sync_copy(ibuf, idxs_ref.at[wid])
    pltpu.sync_copy(mbuf, mxse_ref.at[wid])


@functools.lru_cache(maxsize=1)
def _sc_topk():
    return pl.kernel(
        _sc_topk_body,
        mesh=plsc.VectorSubcoreMesh(core_axis_name="c",
                                    subcore_axis_name="s"),
        out_type=[
            jax.ShapeDtypeStruct((NWORK, 3 * LANES), jnp.float32),
            jax.ShapeDtypeStruct((NWORK, 3 * LANES), jnp.int32),
            jax.ShapeDtypeStruct((NWORK, 2 * LANES), jnp.float32),
        ],
        scratch_types=[
            pltpu.VMEM((CHUNK,), jnp.float32),
            pltpu.VMEM((3 * LANES,), jnp.float32),
            pltpu.VMEM((3 * LANES,), jnp.int32),
            pltpu.VMEM((2 * LANES,), jnp.float32),
        ],
    )


def _merge_body(vals_ref, idxs_ref, mxse_ref, save_id_ref, bz_ref,
                save_out_ref, prob_ref, ids_ref, max_ref):
    v = vals_ref[...]
    ia = idxs_ref[...]
    mx = mxse_ref[:, 0:LANES]
    se = mxse_ref[:, LANES:2 * LANES]
    m = jnp.max(mx)
    ssum = jnp.sum(se * jnp.exp(mx - m))
    lse = jnp.log(ssum) + m
    neg_inf = jnp.float32(-jnp.inf)
    cur = v
    vals, idxs = [], []
    for _ in range(save_id_ref.shape[1]):
        vj = jnp.max(cur)
        hit = cur >= vj
        ij = jnp.min(jnp.where(hit, ia, VOCAB))
        vals.append(vj.reshape(1, 1))
        idxs.append(ij.reshape(1, 1))
        cur = jnp.where(hit & (ia == ij), neg_inf, cur)
    ids_row = jnp.concatenate(idxs, axis=1) + bz_ref[0, 0]
    prob_row = jnp.concatenate(vals, axis=1) - lse
    ids_ref[...] = ids_row
    prob_ref[...] = prob_row
    save_out_ref[0:1, :] = save_id_ref[...]
    save_out_ref[1:2, :] = ids_row
    max_ref[...] = ids_row[:, 0:1]


def kernel(kv0, kv1, kv2, kv3, kv4, kv5, kv6, kv7, kv8, kv9, kv10, kv11,
           kv12, kv13, kv14, kv15, kv16, kv17, kv18, kv19, kv20, kv21,
           kv22, kv23, logits, save_id, beam_size):
    kvs = [kv0, kv1, kv2, kv3, kv4, kv5, kv6, kv7, kv8, kv9, kv10, kv11,
           kv12, kv13, kv14, kv15, kv16, kv17, kv18, kv19, kv20, kv21,
           kv22, kv23]
    beam = save_id.shape[0]

    logits_padded = jnp.concatenate(
        [logits.reshape(-1),
         jnp.full((VPAD - VOCAB,), NEG, jnp.float32)])
    sc_vals, sc_idxs, sc_mxse = _sc_topk()(logits_padded)

    flat_kvs = [kv.reshape(H, S, D).swapaxes(1, 2).reshape(ROWS, S)
                for kv in kvs]
    grid = (ROWS // ROW_CHUNK,)
    in_spec = pl.BlockSpec((ROW_CHUNK, S), lambda i: (i, 0))
    out_spec = pl.BlockSpec((beam, ROW_CHUNK, S), lambda i: (0, i, 0))
    tiled_flat = pl.pallas_call(
        _copy_body,
        grid=grid,
        in_specs=[in_spec] * NUM_LAYERS,
        out_specs=[out_spec] * NUM_LAYERS,
        out_shape=[jax.ShapeDtypeStruct((beam, ROWS, S), kv.dtype)
                   for kv in kvs],
    )(*flat_kvs)
    tiled = [t.reshape(beam, H, D, S).swapaxes(2, 3) for t in tiled_flat]

    bz = (jnp.asarray(beam_size, jnp.int32) - jnp.int32(beam)).reshape(1, 1)
    save_out_t, prob_t, ids_t, max_idx = pl.pallas_call(
        _merge_body,
        in_specs=[
            pl.BlockSpec((NWORK, 3 * LANES), lambda: (0, 0)),
            pl.BlockSpec((NWORK, 3 * LANES), lambda: (0, 0)),
            pl.BlockSpec((NWORK, 2 * LANES), lambda: (0, 0)),
            pl.BlockSpec((1, beam), lambda: (0, 0)),
            pl.BlockSpec((1, 1), lambda: (0, 0)),
        ],
        out_specs=[
            pl.BlockSpec((2, beam), lambda: (0, 0)),
            pl.BlockSpec((1, beam), lambda: (0, 0)),
            pl.BlockSpec((1, beam), lambda: (0, 0)),
            pl.BlockSpec((1, 1), lambda: (0, 0)),
        ],
        out_shape=[
            jax.ShapeDtypeStruct((2, beam), jnp.int32),
            jax.ShapeDtypeStruct((1, beam), jnp.float32),
            jax.ShapeDtypeStruct((1, beam), jnp.int32),
            jax.ShapeDtypeStruct((1, 1), jnp.int32),
        ],
    )(sc_vals, sc_idxs, sc_mxse, save_id.T, bz)

    return (*tiled, save_out_t.T, prob_t.T, ids_t.T, max_idx)
